# baseline (speedup 1.0000x reference)
import functools

import jax
import jax.numpy as jnp
from jax import lax
from jax.experimental import pallas as pl
from jax.experimental.pallas import tpu as pltpu
from jax.experimental.pallas import tpu_sc as plsc

N = 10000
E = 320000
IN = 128
D = 64
H = 128
L = 3
SCALE = 1.5

_NC = 2
_NS = 16
_NW = _NC * _NS
_C = 128
_NCH = E // _C
_NT = (_NCH + _NW - 1) // _NW
_NACC = 10240
_RPT = _NACC // _NS
_ZR = 128
_ACCW = 80

_MESH = plsc.VectorSubcoreMesh(core_axis_name="c", subcore_axis_name="s")


def _gelu(x):
    return 0.5 * x * (1.0 + lax.erf(x * 0.7071067811865476))


def _ln(x, g, b):
    mu = jnp.mean(x, axis=-1, keepdims=True)
    var = jnp.mean((x - mu) ** 2, axis=-1, keepdims=True)
    return (x - mu) / jnp.sqrt(var + 1e-5) * g + b


_CBASE = _NCH // _NW
_CMAX = _CBASE + 1


@functools.partial(
    pl.kernel,
    out_type=jax.ShapeDtypeStruct((E, H), jnp.float32),
    mesh=_MESH,
    scratch_types=[
        pltpu.VMEM((_CMAX * _C,), jnp.int32),
        pltpu.VMEM((_CMAX * _C,), jnp.int32),
        pltpu.VMEM((_C, H), jnp.float32),
        pltpu.VMEM((_C, H), jnp.float32),
        pltpu.VMEM((_C, H), jnp.float32),
        pltpu.VMEM((_C, H), jnp.float32),
        pltpu.SemaphoreType.DMA,
        pltpu.SemaphoreType.DMA,
    ],
)
def _sc_gather_pair(a_hbm, b_hbm, src_hbm, dst_hbm, p_hbm,
                    idxs_all, idxd_all, ba0, bb0, ba1, bb1,
                    sem0, sem1):
    c = lax.axis_index("c")
    s = lax.axis_index("s")
    w = s * _NC + c
    cs = w * _CBASE + jnp.minimum(w, _NCH - _NW * _CBASE)
    cnt = jnp.where(w < _NCH - _NW * _CBASE, _CMAX, _CBASE)

    e0 = cs * _C
    pltpu.sync_copy(src_hbm.at[pl.ds(e0, _CBASE * _C)],
                    idxs_all.at[pl.ds(0, _CBASE * _C)])
    pltpu.sync_copy(dst_hbm.at[pl.ds(e0, _CBASE * _C)],
                    idxd_all.at[pl.ds(0, _CBASE * _C)])

    @pl.when(cnt > _CBASE)
    def _():
        pltpu.sync_copy(src_hbm.at[pl.ds(e0 + _CBASE * _C, _C)],
                        idxs_all.at[pl.ds(_CBASE * _C, _C)])
        pltpu.sync_copy(dst_hbm.at[pl.ds(e0 + _CBASE * _C, _C)],
                        idxd_all.at[pl.ds(_CBASE * _C, _C)])

    def fire(t, ba, bb, sem):
        ia = idxs_all.at[pl.ds(t * _C, _C)]
        ib = idxd_all.at[pl.ds(t * _C, _C)]
        ca = pltpu.async_copy(a_hbm.at[ia], ba, sem)
        cb = pltpu.async_copy(b_hbm.at[ib], bb, sem)
        return ca, cb

    def finish(t, ba, bb, copies):
        for cc in copies:
            cc.wait()

        def row(r, carry2):
            ra = ba.at[r]
            rb = bb.at[r]
            for i in range(H // 16):
                sl = pl.ds(i * 16, 16)
                ra[sl] = ra[sl] + rb[sl]
            return carry2

        lax.fori_loop(0, _C, row, 0)
        base = (cs + t) * _C
        pltpu.sync_copy(ba, p_hbm.at[pl.ds(base, _C)])

    def pair(j, carry):
        t0 = 2 * j
        t1 = 2 * j + 1

        @pl.when(t1 < cnt)
        def _():
            c0 = fire(t0, ba0, bb0, sem0)
            c1 = fire(t1, ba1, bb1, sem1)
            finish(t0, ba0, bb0, c0)
            finish(t1, ba1, bb1, c1)

        @pl.when((t0 < cnt) & (t1 >= cnt))
        def _():
            c0 = fire(t0, ba0, bb0, sem0)
            finish(t0, ba0, bb0, c0)

        return carry

    lax.fori_loop(0, (_CMAX + 1) // 2, pair, 0)


@functools.partial(
    pl.kernel,
    out_type=jax.ShapeDtypeStruct((_NC, _NACC, H), jnp.float32),
    mesh=_MESH,
    scratch_types=[
        pltpu.VMEM((_C,), jnp.int32),
        pltpu.VMEM((_C,), jnp.int32),
        pltpu.VMEM((_C,), jnp.float32),
        pltpu.VMEM((_C, H), jnp.float32),
        pltpu.VMEM_SHARED((_NACC, H), jnp.float32),
        pltpu.SemaphoreType.DMA,
    ],
)
def _sc_scatter(h_hbm, gate_hbm, src_hbm, dst_hbm, out_hbm,
                idxs, idxd, gbuf, hbuf, acc, sem):
    c = lax.axis_index("c")
    s = lax.axis_index("s")
    w = s * _NC + c

    def zrow(r, carry):
        rz = hbuf.at[r]
        for i in range(H // 16):
            rz[pl.ds(i * 16, 16)] = jnp.zeros((16,), jnp.float32)
        return carry

    lax.fori_loop(0, _ZR, zrow, 0)
    for t in range(_RPT // _ZR):
        pltpu.sync_copy(hbuf, acc.at[pl.ds(s * _RPT + t * _ZR, _ZR)])
    plsc.subcore_barrier()

    def step(t, carry):
        cid = w + t * _NW

        @pl.when(cid < _NCH)
        def _():
            base = cid * _C
            pltpu.sync_copy(src_hbm.at[pl.ds(base, _C)], idxs)
            pltpu.sync_copy(dst_hbm.at[pl.ds(base, _C)], idxd)
            pltpu.sync_copy(gate_hbm.at[pl.ds(base, _C)], gbuf)
            pltpu.async_copy(h_hbm.at[idxs], hbuf, sem).wait()

            def egroup(g, carry2):
                gv = gbuf[pl.ds(g * 16, 16)]
                for k in range(16):
                    g16 = jnp.full((16,), gv[k], jnp.float32)
                    rh = hbuf.at[g * 16 + k]
                    for j in range(H // 16):
                        sl = pl.ds(j * 16, 16)
                        rh[sl] = rh[sl] * g16
                return carry2

            lax.fori_loop(0, _C // 16, egroup, 0)
            pltpu.sync_copy(hbuf, acc.at[idxd], add=True)

        return carry

    lax.fori_loop(0, _NT, step, 0)
    plsc.subcore_barrier()

    for t in range(_RPT // _ZR):
        base = s * _RPT + t * _ZR
        pltpu.sync_copy(acc.at[pl.ds(base, _ZR)],
                        out_hbm.at[c, pl.ds(base, _ZR)])


_BN = 2000
_BE = 8000


def _enc_body(x_ref, w1, b1, g, be, w2, b2, w1s, w1d, eb1,
              h_ref, a_ref, b_ref):
    h = jnp.dot(x_ref[...], w1[...], preferred_element_type=jnp.float32) + b1[...]
    h = _ln(h, g[...], be[...])
    h = _gelu(h)
    h = jnp.dot(h, w2[...], preferred_element_type=jnp.float32) + b2[...]
    h_ref[...] = jnp.concatenate(
        [h, jnp.ones((h.shape[0], H - D), jnp.float32)], axis=1)
    a_ref[...] = jnp.dot(h, w1s[...], preferred_element_type=jnp.float32) + eb1[...]
    b_ref[...] = jnp.dot(h, w1d[...], preferred_element_type=jnp.float32)


def _encoder(x, w1, b1, g, be, w2, b2, w1s, w1d, eb1):
    grid = (N // _BN,)
    full = lambda shp: pl.BlockSpec(shp, lambda i: (0, 0))
    return pl.pallas_call(
        _enc_body,
        grid=grid,
        in_specs=[
            pl.BlockSpec((_BN, IN), lambda i: (i, 0)),
            full((IN, D)), full((1, D)), full((1, D)), full((1, D)),
            full((D, D)), full((1, D)),
            full((D, H)), full((D, H)), full((1, H)),
        ],
        out_specs=[
            pl.BlockSpec((_BN, H), lambda i: (i, 0)),
            pl.BlockSpec((_BN, H), lambda i: (i, 0)),
            pl.BlockSpec((_BN, H), lambda i: (i, 0)),
        ],
        out_shape=[
            jax.ShapeDtypeStruct((N, H), jnp.float32),
            jax.ShapeDtypeStruct((N, H), jnp.float32),
            jax.ShapeDtypeStruct((N, H), jnp.float32),
        ],
    )(x, w1, b1, g, be, w2, b2, w1s, w1d, eb1)


def _gate_body(p_ref, ea_ref, w1e, w2, b2, gate_ref):
    pre = p_ref[...] + ea_ref[...] * w1e[...]
    gg = _gelu(pre)
    z = jnp.dot(gg, w2[...], preferred_element_type=jnp.float32) + b2[...]
    gate_ref[...] = jax.nn.sigmoid(z)


def _edge_gate(p, ea, w1e, w2, b2):
    grid = (E // _BE,)
    full = lambda shp: pl.BlockSpec(shp, lambda i: (0, 0))
    return pl.pallas_call(
        _gate_body,
        grid=grid,
        in_specs=[
            pl.BlockSpec((_BE, H), lambda i: (i, 0)),
            pl.BlockSpec((_BE, 1), lambda i: (i, 0)),
            full((1, H)), full((H, 1)), full((1, 1)),
        ],
        out_specs=pl.BlockSpec((_BE, 1), lambda i: (i, 0)),
        out_shape=jax.ShapeDtypeStruct((E, 1), jnp.float32),
    )(p, ea, w1e, w2, b2)


def _node_body(h_ref, p0_ref, p1_ref, w1h, w1a, b1, w2, b2, lng, lnb,
               nw1s, nw1d, neb1, h_out, a_out, b_out):
    part = p0_ref[...] + p1_ref[...]
    agg = part[:, :D]
    den = part[:, D:D + 1]
    agg = agg / jnp.maximum(den, 1e-6)
    h = h_ref[:, :D]
    u = _gelu(jnp.dot(h, w1h[...], preferred_element_type=jnp.float32)
              + jnp.dot(agg, w1a[...], preferred_element_type=jnp.float32)
              + b1[...])
    u = jnp.dot(u, w2[...], preferred_element_type=jnp.float32) + b2[...]
    hn = _ln(h + u, lng[...], lnb[...])
    h_out[...] = jnp.concatenate(
        [hn, jnp.ones((hn.shape[0], H - D), jnp.float32)], axis=1)
    a_out[...] = jnp.dot(hn, nw1s[...], preferred_element_type=jnp.float32) + neb1[...]
    b_out[...] = jnp.dot(hn, nw1d[...], preferred_element_type=jnp.float32)


def _node_update(h, p0, p1, w1h, w1a, b1, w2, b2, lng, lnb, nw1s, nw1d, neb1):
    grid = (N // _BN,)
    full = lambda shp: pl.BlockSpec(shp, lambda i: (0, 0))
    return pl.pallas_call(
        _node_body,
        grid=grid,
        in_specs=[
            pl.BlockSpec((_BN, H), lambda i: (i, 0)),
            pl.BlockSpec((_BN, H), lambda i: (i, 0)),
            pl.BlockSpec((_BN, H), lambda i: (i, 0)),
            full((D, H)), full((D, H)), full((1, H)),
            full((H, D)), full((1, D)), full((1, D)), full((1, D)),
            full((D, H)), full((D, H)), full((1, H)),
        ],
        out_specs=[
            pl.BlockSpec((_BN, H), lambda i: (i, 0)),
            pl.BlockSpec((_BN, H), lambda i: (i, 0)),
            pl.BlockSpec((_BN, H), lambda i: (i, 0)),
        ],
        out_shape=[
            jax.ShapeDtypeStruct((N, H), jnp.float32),
            jax.ShapeDtypeStruct((N, H), jnp.float32),
            jax.ShapeDtypeStruct((N, H), jnp.float32),
        ],
    )(h, p0, p1, w1h, w1a, b1, w2, b2, lng, lnb, nw1s, nw1d, neb1)


def _head_body(h_ref, p0_ref, p1_ref, w1h, w1a, b1, w2, b2, lng, lnb,
               hw1, hb1, hw2, hb2, out_ref):
    part = p0_ref[...] + p1_ref[...]
    agg = part[:, :D]
    den = part[:, D:D + 1]
    agg = agg / jnp.maximum(den, 1e-6)
    h = h_ref[:, :D]
    u = _gelu(jnp.dot(h, w1h[...], preferred_element_type=jnp.float32)
              + jnp.dot(agg, w1a[...], preferred_element_type=jnp.float32)
              + b1[...])
    u = jnp.dot(u, w2[...], preferred_element_type=jnp.float32) + b2[...]
    hn = _ln(h + u, lng[...], lnb[...])
    raw = jnp.dot(_gelu(jnp.dot(hn, hw1[...], preferred_element_type=jnp.float32)
                        + hb1[...]),
                  hw2[...], preferred_element_type=jnp.float32) + hb2[...]
    out_ref[...] = 1.0 + SCALE * jnp.tanh(raw)


def _head(h, p0, p1, w1h, w1a, b1, w2, b2, lng, lnb, hw1, hb1, hw2, hb2):
    grid = (N // _BN,)
    full = lambda shp: pl.BlockSpec(shp, lambda i: (0, 0))
    return pl.pallas_call(
        _head_body,
        grid=grid,
        in_specs=[
            pl.BlockSpec((_BN, H), lambda i: (i, 0)),
            pl.BlockSpec((_BN, H), lambda i: (i, 0)),
            pl.BlockSpec((_BN, H), lambda i: (i, 0)),
            full((D, H)), full((D, H)), full((1, H)),
            full((H, D)), full((1, D)), full((1, D)), full((1, D)),
            full((D, H)), full((1, H)), full((H, 1)), full((1, 1)),
        ],
        out_specs=pl.BlockSpec((_BN, 1), lambda i: (i, 0)),
        out_shape=jax.ShapeDtypeStruct((N, 1), jnp.float32),
    )(h, p0, p1, w1h, w1a, b1, w2, b2, lng, lnb, hw1, hb1, hw2, hb2)


def kernel(x, edge_index, edge_attr, params):
    p = params
    src = jnp.asarray(edge_index[:, 0], jnp.int32)
    dst = jnp.asarray(edge_index[:, 1], jnp.int32)

    def r2(v, n):
        return v.reshape(1, n)

    esplit = []
    for i in range(L):
        w1 = p[f'e{i}_w1']
        esplit.append((w1[:D], w1[D:2 * D], w1[2 * D:2 * D + 1],
                       r2(p[f'e{i}_b1'], H)))

    h, A, B = _encoder(
        x, p['enc_w1'], r2(p['enc_b1'], D), r2(p['enc_g'], D),
        r2(p['enc_be'], D), p['enc_w2'], r2(p['enc_b2'], D),
        esplit[0][0], esplit[0][1], esplit[0][3])

    out = None
    for i in range(L):
        P = _sc_gather_pair(A, B, src, dst)
        gate = _edge_gate(P, edge_attr, esplit[i][2], p[f'e{i}_w2'],
                          r2(p[f'e{i}_b2'], 1))
        parts = _sc_scatter(h, gate.reshape(E), src, dst)
        nw1 = p[f'n{i}_w1']
        common = (h, parts[0], parts[1], nw1[:D], nw1[D:], r2(p[f'n{i}_b1'], H),
                  p[f'n{i}_w2'], r2(p[f'n{i}_b2'], D),
                  r2(p[f'ln{i}_g'], D), r2(p[f'ln{i}_b'], D))
        if i < L - 1:
            h, A, B = _node_update(*common, esplit[i + 1][0], esplit[i + 1][1],
                                   esplit[i + 1][3])
        else:
            out = _head(*common, p['h_w1'], r2(p['h_b1'], H),
                        p['h_w2'], r2(p['h_b2'], 1))
    return out.reshape(N)


# speedup vs baseline: 1.1229x; 1.1229x over previous
import functools

import jax
import jax.numpy as jnp
from jax import lax
from jax.experimental import pallas as pl
from jax.experimental.pallas import tpu as pltpu
from jax.experimental.pallas import tpu_sc as plsc

N = 10000
E = 320000
IN = 128
D = 64
H = 128
L = 3
SCALE = 1.5

_NC = 2
_NS = 16
_NW = _NC * _NS
_C = 128
_NCH = E // _C
_NT = (_NCH + _NW - 1) // _NW
_NACC = 10240
_RPT = _NACC // _NS
_ZR = 128
_ACCW = 128

_MESH = plsc.VectorSubcoreMesh(core_axis_name="c", subcore_axis_name="s")


def _gelu(x):
    return 0.5 * x * (1.0 + lax.erf(x * 0.7071067811865476))


def _ln(x, g, b):
    mu = jnp.mean(x, axis=-1, keepdims=True)
    var = jnp.mean((x - mu) ** 2, axis=-1, keepdims=True)
    return (x - mu) / jnp.sqrt(var + 1e-5) * g + b


_CBASE = _NCH // _NW
_CMAX = _CBASE + 1


@functools.partial(
    pl.kernel,
    out_type=jax.ShapeDtypeStruct((E, H), jnp.float32),
    mesh=_MESH,
    scratch_types=[
        pltpu.VMEM((_CMAX * _C,), jnp.int32),
        pltpu.VMEM((_CMAX * _C,), jnp.int32),
        pltpu.VMEM((_C, H), jnp.float32),
        pltpu.VMEM((_C, H), jnp.float32),
        pltpu.VMEM((_C, H), jnp.float32),
        pltpu.VMEM((_C, H), jnp.float32),
        pltpu.SemaphoreType.DMA,
        pltpu.SemaphoreType.DMA,
    ],
)
def _sc_gather_pair(a_hbm, b_hbm, src_hbm, dst_hbm, p_hbm,
                    idxs_all, idxd_all, ba0, bb0, ba1, bb1,
                    sem0, sem1):
    c = lax.axis_index("c")
    s = lax.axis_index("s")
    w = s * _NC + c
    cs = w * _CBASE + jnp.minimum(w, _NCH - _NW * _CBASE)
    cnt = jnp.where(w < _NCH - _NW * _CBASE, _CMAX, _CBASE)

    e0 = cs * _C
    pltpu.sync_copy(src_hbm.at[pl.ds(e0, _CBASE * _C)],
                    idxs_all.at[pl.ds(0, _CBASE * _C)])
    pltpu.sync_copy(dst_hbm.at[pl.ds(e0, _CBASE * _C)],
                    idxd_all.at[pl.ds(0, _CBASE * _C)])

    @pl.when(cnt > _CBASE)
    def _():
        pltpu.sync_copy(src_hbm.at[pl.ds(e0 + _CBASE * _C, _C)],
                        idxs_all.at[pl.ds(_CBASE * _C, _C)])
        pltpu.sync_copy(dst_hbm.at[pl.ds(e0 + _CBASE * _C, _C)],
                        idxd_all.at[pl.ds(_CBASE * _C, _C)])

    def fire(t, ba, bb, sem):
        ia = idxs_all.at[pl.ds(t * _C, _C)]
        ib = idxd_all.at[pl.ds(t * _C, _C)]
        ca = pltpu.async_copy(a_hbm.at[ia], ba, sem)
        cb = pltpu.async_copy(b_hbm.at[ib], bb, sem)
        return ca, cb

    def finish(t, ba, bb, copies):
        for cc in copies:
            cc.wait()

        def row(r, carry2):
            ra = ba.at[r]
            rb = bb.at[r]
            for i in range(H // 16):
                sl = pl.ds(i * 16, 16)
                ra[sl] = ra[sl] + rb[sl]
            return carry2

        lax.fori_loop(0, _C, row, 0)
        base = (cs + t) * _C
        pltpu.sync_copy(ba, p_hbm.at[pl.ds(base, _C)])

    def pair(j, carry):
        t0 = 2 * j
        t1 = 2 * j + 1

        @pl.when(t1 < cnt)
        def _():
            c0 = fire(t0, ba0, bb0, sem0)
            c1 = fire(t1, ba1, bb1, sem1)
            finish(t0, ba0, bb0, c0)
            finish(t1, ba1, bb1, c1)

        @pl.when((t0 < cnt) & (t1 >= cnt))
        def _():
            c0 = fire(t0, ba0, bb0, sem0)
            finish(t0, ba0, bb0, c0)

        return carry

    lax.fori_loop(0, (_CMAX + 1) // 2, pair, 0)


@functools.partial(
    pl.kernel,
    out_type=jax.ShapeDtypeStruct((_NC, _NACC, _ACCW), jnp.float32),
    mesh=_MESH,
    scratch_types=[
        pltpu.VMEM((_C,), jnp.int32),
        pltpu.VMEM((_C,), jnp.int32),
        pltpu.VMEM((_C,), jnp.float32),
        pltpu.VMEM((_C, H), jnp.float32),
        pltpu.VMEM((_C,), jnp.int32),
        pltpu.VMEM((_C,), jnp.int32),
        pltpu.VMEM((_C,), jnp.float32),
        pltpu.VMEM((_C, H), jnp.float32),
        pltpu.VMEM_SHARED((_NACC, _ACCW), jnp.float32),
        pltpu.SemaphoreType.DMA,
        pltpu.SemaphoreType.DMA,
    ],
)
def _sc_scatter(h_hbm, gate_hbm, src_hbm, dst_hbm, out_hbm,
                idxs0, idxd0, gbuf0, hbuf0,
                idxs1, idxd1, gbuf1, hbuf1, acc, sem0, sem1):
    c = lax.axis_index("c")
    s = lax.axis_index("s")
    w = s * _NC + c

    def zrow(r, carry):
        rz = hbuf0.at[r]
        for i in range(_ACCW // 16):
            rz[pl.ds(i * 16, 16)] = jnp.zeros((16,), jnp.float32)
        return carry

    lax.fori_loop(0, _ZR, zrow, 0)
    for t in range(_RPT // _ZR):
        pltpu.sync_copy(hbuf0, acc.at[pl.ds(s * _RPT + t * _ZR, _ZR)])
    plsc.subcore_barrier()

    def fire(cid, idxs, idxd, gbuf, hbuf, sem):
        base = cid * _C
        pltpu.sync_copy(src_hbm.at[pl.ds(base, _C)], idxs)
        pltpu.sync_copy(dst_hbm.at[pl.ds(base, _C)], idxd)
        pltpu.sync_copy(gate_hbm.at[pl.ds(base, _C)], gbuf)
        return pltpu.async_copy(h_hbm.at[idxs], hbuf, sem)

    def finish(idxd, gbuf, hbuf, copy):
        copy.wait()

        def egroup(g, carry2):
            gv = gbuf[pl.ds(g * 16, 16)]
            for k in range(16):
                g16 = jnp.full((16,), gv[k], jnp.float32)
                rh = hbuf.at[g * 16 + k]
                for j in range(_ACCW // 16):
                    sl = pl.ds(j * 16, 16)
                    rh[sl] = rh[sl] * g16
            return carry2

        lax.fori_loop(0, _C // 16, egroup, 0)
        pltpu.sync_copy(hbuf, acc.at[idxd], add=True)

    def pair(j, carry):
        cid0 = w + (2 * j) * _NW
        cid1 = w + (2 * j + 1) * _NW

        @pl.when(cid1 < _NCH)
        def _():
            c0 = fire(cid0, idxs0, idxd0, gbuf0, hbuf0, sem0)
            c1 = fire(cid1, idxs1, idxd1, gbuf1, hbuf1, sem1)
            finish(idxd0, gbuf0, hbuf0, c0)
            finish(idxd1, gbuf1, hbuf1, c1)

        @pl.when((cid0 < _NCH) & (cid1 >= _NCH))
        def _():
            c0 = fire(cid0, idxs0, idxd0, gbuf0, hbuf0, sem0)
            finish(idxd0, gbuf0, hbuf0, c0)

        return carry

    lax.fori_loop(0, (_NT + 1) // 2, pair, 0)
    plsc.subcore_barrier()

    for t in range(_RPT // _ZR):
        base = s * _RPT + t * _ZR
        pltpu.sync_copy(acc.at[pl.ds(base, _ZR)],
                        out_hbm.at[c, pl.ds(base, _ZR)])


_BN = 2000
_BE = 8000


def _enc_body(x_ref, w1, b1, g, be, w2, b2, w1s, w1d, eb1,
              h_ref, a_ref, b_ref):
    h = jnp.dot(x_ref[...], w1[...], preferred_element_type=jnp.float32) + b1[...]
    h = _ln(h, g[...], be[...])
    h = _gelu(h)
    h = jnp.dot(h, w2[...], preferred_element_type=jnp.float32) + b2[...]
    h_ref[...] = jnp.concatenate(
        [h, jnp.ones((h.shape[0], H - D), jnp.float32)], axis=1)
    a_ref[...] = jnp.dot(h, w1s[...], preferred_element_type=jnp.float32) + eb1[...]
    b_ref[...] = jnp.dot(h, w1d[...], preferred_element_type=jnp.float32)


def _encoder(x, w1, b1, g, be, w2, b2, w1s, w1d, eb1):
    grid = (N // _BN,)
    full = lambda shp: pl.BlockSpec(shp, lambda i: (0, 0))
    return pl.pallas_call(
        _enc_body,
        grid=grid,
        in_specs=[
            pl.BlockSpec((_BN, IN), lambda i: (i, 0)),
            full((IN, D)), full((1, D)), full((1, D)), full((1, D)),
            full((D, D)), full((1, D)),
            full((D, H)), full((D, H)), full((1, H)),
        ],
        out_specs=[
            pl.BlockSpec((_BN, H), lambda i: (i, 0)),
            pl.BlockSpec((_BN, H), lambda i: (i, 0)),
            pl.BlockSpec((_BN, H), lambda i: (i, 0)),
        ],
        out_shape=[
            jax.ShapeDtypeStruct((N, H), jnp.float32),
            jax.ShapeDtypeStruct((N, H), jnp.float32),
            jax.ShapeDtypeStruct((N, H), jnp.float32),
        ],
    )(x, w1, b1, g, be, w2, b2, w1s, w1d, eb1)


def _gate_body(p_ref, ea_ref, w1e, w2, b2, gate_ref):
    pre = p_ref[...] + ea_ref[...] * w1e[...]
    gg = _gelu(pre)
    z = jnp.dot(gg, w2[...], preferred_element_type=jnp.float32) + b2[...]
    gate_ref[...] = jax.nn.sigmoid(z)


def _edge_gate(p, ea, w1e, w2, b2):
    grid = (E // _BE,)
    full = lambda shp: pl.BlockSpec(shp, lambda i: (0, 0))
    return pl.pallas_call(
        _gate_body,
        grid=grid,
        in_specs=[
            pl.BlockSpec((_BE, H), lambda i: (i, 0)),
            pl.BlockSpec((_BE, 1), lambda i: (i, 0)),
            full((1, H)), full((H, 1)), full((1, 1)),
        ],
        out_specs=pl.BlockSpec((_BE, 1), lambda i: (i, 0)),
        out_shape=jax.ShapeDtypeStruct((E, 1), jnp.float32),
    )(p, ea, w1e, w2, b2)


def _node_body(h_ref, p0_ref, p1_ref, w1h, w1a, b1, w2, b2, lng, lnb,
               nw1s, nw1d, neb1, h_out, a_out, b_out):
    part = p0_ref[...] + p1_ref[...]
    agg = part[:, :D]
    den = part[:, D:D + 1]
    agg = agg / jnp.maximum(den, 1e-6)
    h = h_ref[:, :D]
    u = _gelu(jnp.dot(h, w1h[...], preferred_element_type=jnp.float32)
              + jnp.dot(agg, w1a[...], preferred_element_type=jnp.float32)
              + b1[...])
    u = jnp.dot(u, w2[...], preferred_element_type=jnp.float32) + b2[...]
    hn = _ln(h + u, lng[...], lnb[...])
    h_out[...] = jnp.concatenate(
        [hn, jnp.ones((hn.shape[0], H - D), jnp.float32)], axis=1)
    a_out[...] = jnp.dot(hn, nw1s[...], preferred_element_type=jnp.float32) + neb1[...]
    b_out[...] = jnp.dot(hn, nw1d[...], preferred_element_type=jnp.float32)


def _node_update(h, p0, p1, w1h, w1a, b1, w2, b2, lng, lnb, nw1s, nw1d, neb1):
    grid = (N // _BN,)
    full = lambda shp: pl.BlockSpec(shp, lambda i: (0, 0))
    return pl.pallas_call(
        _node_body,
        grid=grid,
        in_specs=[
            pl.BlockSpec((_BN, H), lambda i: (i, 0)),
            pl.BlockSpec((_BN, _ACCW), lambda i: (i, 0)),
            pl.BlockSpec((_BN, _ACCW), lambda i: (i, 0)),
            full((D, H)), full((D, H)), full((1, H)),
            full((H, D)), full((1, D)), full((1, D)), full((1, D)),
            full((D, H)), full((D, H)), full((1, H)),
        ],
        out_specs=[
            pl.BlockSpec((_BN, H), lambda i: (i, 0)),
            pl.BlockSpec((_BN, H), lambda i: (i, 0)),
            pl.BlockSpec((_BN, H), lambda i: (i, 0)),
        ],
        out_shape=[
            jax.ShapeDtypeStruct((N, H), jnp.float32),
            jax.ShapeDtypeStruct((N, H), jnp.float32),
            jax.ShapeDtypeStruct((N, H), jnp.float32),
        ],
    )(h, p0, p1, w1h, w1a, b1, w2, b2, lng, lnb, nw1s, nw1d, neb1)


def _head_body(h_ref, p0_ref, p1_ref, w1h, w1a, b1, w2, b2, lng, lnb,
               hw1, hb1, hw2, hb2, out_ref):
    part = p0_ref[...] + p1_ref[...]
    agg = part[:, :D]
    den = part[:, D:D + 1]
    agg = agg / jnp.maximum(den, 1e-6)
    h = h_ref[:, :D]
    u = _gelu(jnp.dot(h, w1h[...], preferred_element_type=jnp.float32)
              + jnp.dot(agg, w1a[...], preferred_element_type=jnp.float32)
              + b1[...])
    u = jnp.dot(u, w2[...], preferred_element_type=jnp.float32) + b2[...]
    hn = _ln(h + u, lng[...], lnb[...])
    raw = jnp.dot(_gelu(jnp.dot(hn, hw1[...], preferred_element_type=jnp.float32)
                        + hb1[...]),
                  hw2[...], preferred_element_type=jnp.float32) + hb2[...]
    out_ref[...] = 1.0 + SCALE * jnp.tanh(raw)


def _head(h, p0, p1, w1h, w1a, b1, w2, b2, lng, lnb, hw1, hb1, hw2, hb2):
    grid = (N // _BN,)
    full = lambda shp: pl.BlockSpec(shp, lambda i: (0, 0))
    return pl.pallas_call(
        _head_body,
        grid=grid,
        in_specs=[
            pl.BlockSpec((_BN, H), lambda i: (i, 0)),
            pl.BlockSpec((_BN, _ACCW), lambda i: (i, 0)),
            pl.BlockSpec((_BN, _ACCW), lambda i: (i, 0)),
            full((D, H)), full((D, H)), full((1, H)),
            full((H, D)), full((1, D)), full((1, D)), full((1, D)),
            full((D, H)), full((1, H)), full((H, 1)), full((1, 1)),
        ],
        out_specs=pl.BlockSpec((_BN, 1), lambda i: (i, 0)),
        out_shape=jax.ShapeDtypeStruct((N, 1), jnp.float32),
    )(h, p0, p1, w1h, w1a, b1, w2, b2, lng, lnb, hw1, hb1, hw2, hb2)


def kernel(x, edge_index, edge_attr, params):
    p = params
    src = jnp.asarray(edge_index[:, 0], jnp.int32)
    dst = jnp.asarray(edge_index[:, 1], jnp.int32)

    def r2(v, n):
        return v.reshape(1, n)

    esplit = []
    for i in range(L):
        w1 = p[f'e{i}_w1']
        esplit.append((w1[:D], w1[D:2 * D], w1[2 * D:2 * D + 1],
                       r2(p[f'e{i}_b1'], H)))

    h, A, B = _encoder(
        x, p['enc_w1'], r2(p['enc_b1'], D), r2(p['enc_g'], D),
        r2(p['enc_be'], D), p['enc_w2'], r2(p['enc_b2'], D),
        esplit[0][0], esplit[0][1], esplit[0][3])

    out = None
    for i in range(L):
        P = _sc_gather_pair(A, B, src, dst)
        gate = _edge_gate(P, edge_attr, esplit[i][2], p[f'e{i}_w2'],
                          r2(p[f'e{i}_b2'], 1))
        parts = _sc_scatter(h, gate.reshape(E), src, dst)
        nw1 = p[f'n{i}_w1']
        common = (h, parts[0], parts[1], nw1[:D], nw1[D:], r2(p[f'n{i}_b1'], H),
                  p[f'n{i}_w2'], r2(p[f'n{i}_b2'], D),
                  r2(p[f'ln{i}_g'], D), r2(p[f'ln{i}_b'], D))
        if i < L - 1:
            h, A, B = _node_update(*common, esplit[i + 1][0], esplit[i + 1][1],
                                   esplit[i + 1][3])
        else:
            out = _head(*common, p['h_w1'], r2(p['h_b1'], H),
                        p['h_w2'], r2(p['h_b2'], 1))
    return out.reshape(N)


# speedup vs baseline: 1.1277x; 1.0043x over previous
import functools

import jax
import jax.numpy as jnp
from jax import lax
from jax.experimental import pallas as pl
from jax.experimental.pallas import tpu as pltpu
from jax.experimental.pallas import tpu_sc as plsc

N = 10000
E = 320000
IN = 128
D = 64
H = 128
L = 3
SCALE = 1.5

_NC = 2
_NS = 16
_NW = _NC * _NS
_C = 128
_NCH = E // _C
_NT = (_NCH + _NW - 1) // _NW
_NACC = 10240
_RPT = _NACC // _NS
_ZR = 128
_ACCW = 128

_MESH = plsc.VectorSubcoreMesh(core_axis_name="c", subcore_axis_name="s")


def _gelu(x):
    return 0.5 * x * (1.0 + lax.erf(x * 0.7071067811865476))


def _ln(x, g, b):
    mu = jnp.mean(x, axis=-1, keepdims=True)
    var = jnp.mean((x - mu) ** 2, axis=-1, keepdims=True)
    return (x - mu) / jnp.sqrt(var + 1e-5) * g + b


_CBASE = _NCH // _NW
_CMAX = _CBASE + 1


@functools.partial(
    pl.kernel,
    out_type=jax.ShapeDtypeStruct((E, H), jnp.float32),
    mesh=_MESH,
    scratch_types=[
        pltpu.VMEM((_CMAX * _C,), jnp.int32),
        pltpu.VMEM((_CMAX * _C,), jnp.int32),
        pltpu.VMEM((_C, H), jnp.float32),
        pltpu.VMEM((_C, H), jnp.float32),
        pltpu.VMEM((_C, H), jnp.float32),
        pltpu.VMEM((_C, H), jnp.float32),
        pltpu.SemaphoreType.DMA,
        pltpu.SemaphoreType.DMA,
    ],
)
def _sc_gather_pair(a_hbm, b_hbm, src_hbm, dst_hbm, p_hbm,
                    idxs_all, idxd_all, ba0, bb0, ba1, bb1,
                    sem0, sem1):
    c = lax.axis_index("c")
    s = lax.axis_index("s")
    w = s * _NC + c
    cs = w * _CBASE + jnp.minimum(w, _NCH - _NW * _CBASE)
    cnt = jnp.where(w < _NCH - _NW * _CBASE, _CMAX, _CBASE)

    e0 = cs * _C
    pltpu.sync_copy(src_hbm.at[pl.ds(e0, _CBASE * _C)],
                    idxs_all.at[pl.ds(0, _CBASE * _C)])
    pltpu.sync_copy(dst_hbm.at[pl.ds(e0, _CBASE * _C)],
                    idxd_all.at[pl.ds(0, _CBASE * _C)])

    @pl.when(cnt > _CBASE)
    def _():
        pltpu.sync_copy(src_hbm.at[pl.ds(e0 + _CBASE * _C, _C)],
                        idxs_all.at[pl.ds(_CBASE * _C, _C)])
        pltpu.sync_copy(dst_hbm.at[pl.ds(e0 + _CBASE * _C, _C)],
                        idxd_all.at[pl.ds(_CBASE * _C, _C)])

    def fire(t, ba, bb, sem):
        ia = idxs_all.at[pl.ds(t * _C, _C)]
        ib = idxd_all.at[pl.ds(t * _C, _C)]
        ca = pltpu.async_copy(a_hbm.at[ia], ba, sem)
        cb = pltpu.async_copy(b_hbm.at[ib], bb, sem)
        return ca, cb

    def finish(t, ba, bb, copies):
        for cc in copies:
            cc.wait()

        def row(r, carry2):
            ra = ba.at[r]
            rb = bb.at[r]
            for i in range(H // 16):
                sl = pl.ds(i * 16, 16)
                ra[sl] = ra[sl] + rb[sl]
            return carry2

        lax.fori_loop(0, _C, row, 0)
        base = (cs + t) * _C
        pltpu.sync_copy(ba, p_hbm.at[pl.ds(base, _C)])

    def pair(j, carry):
        t0 = 2 * j
        t1 = 2 * j + 1

        @pl.when(t1 < cnt)
        def _():
            c0 = fire(t0, ba0, bb0, sem0)
            c1 = fire(t1, ba1, bb1, sem1)
            finish(t0, ba0, bb0, c0)
            finish(t1, ba1, bb1, c1)

        @pl.when((t0 < cnt) & (t1 >= cnt))
        def _():
            c0 = fire(t0, ba0, bb0, sem0)
            finish(t0, ba0, bb0, c0)

        return carry

    lax.fori_loop(0, (_CMAX + 1) // 2, pair, 0)


@functools.partial(
    pl.kernel,
    out_type=jax.ShapeDtypeStruct((_NC, _NACC, _ACCW), jnp.float32),
    mesh=_MESH,
    scratch_types=[
        pltpu.VMEM((_C,), jnp.int32),
        pltpu.VMEM((_C,), jnp.int32),
        pltpu.VMEM((_C,), jnp.float32),
        pltpu.VMEM((_C, H), jnp.float32),
        pltpu.VMEM((_C,), jnp.int32),
        pltpu.VMEM((_C,), jnp.int32),
        pltpu.VMEM((_C,), jnp.float32),
        pltpu.VMEM((_C, H), jnp.float32),
        pltpu.VMEM_SHARED((_NACC, _ACCW), jnp.float32),
        pltpu.SemaphoreType.DMA,
        pltpu.SemaphoreType.DMA,
    ],
)
def _sc_scatter(h_hbm, gate_hbm, src_hbm, dst_hbm, out_hbm,
                idxs0, idxd0, gbuf0, hbuf0,
                idxs1, idxd1, gbuf1, hbuf1, acc, sem0, sem1):
    c = lax.axis_index("c")
    s = lax.axis_index("s")
    w = s * _NC + c

    def zrow(r, carry):
        rz = hbuf0.at[r]
        for i in range(_ACCW // 16):
            rz[pl.ds(i * 16, 16)] = jnp.zeros((16,), jnp.float32)
        return carry

    lax.fori_loop(0, _ZR, zrow, 0)
    for t in range(_RPT // _ZR):
        pltpu.sync_copy(hbuf0, acc.at[pl.ds(s * _RPT + t * _ZR, _ZR)])
    plsc.subcore_barrier()

    def fire(cid, idxs, idxd, gbuf, hbuf, sem):
        base = cid * _C
        pltpu.sync_copy(src_hbm.at[pl.ds(base, _C)], idxs)
        pltpu.sync_copy(dst_hbm.at[pl.ds(base, _C)], idxd)
        pltpu.sync_copy(gate_hbm.at[pl.ds(base, _C)], gbuf)
        return pltpu.async_copy(h_hbm.at[idxs], hbuf, sem)

    def finish(idxd, gbuf, hbuf, copy):
        copy.wait()

        def egroup(g, carry2):
            gv = gbuf[pl.ds(g * 16, 16)]
            for k in range(16):
                g16 = jnp.full((16,), gv[k], jnp.float32)
                rh = hbuf.at[g * 16 + k]
                for j in range(_ACCW // 16):
                    sl = pl.ds(j * 16, 16)
                    rh[sl] = rh[sl] * g16
            return carry2

        lax.fori_loop(0, _C // 16, egroup, 0)
        pltpu.sync_copy(hbuf, acc.at[idxd], add=True)

    def pair(j, carry):
        cid0 = w + (2 * j) * _NW
        cid1 = w + (2 * j + 1) * _NW

        @pl.when(cid1 < _NCH)
        def _():
            c0 = fire(cid0, idxs0, idxd0, gbuf0, hbuf0, sem0)
            c1 = fire(cid1, idxs1, idxd1, gbuf1, hbuf1, sem1)
            finish(idxd0, gbuf0, hbuf0, c0)
            finish(idxd1, gbuf1, hbuf1, c1)

        @pl.when((cid0 < _NCH) & (cid1 >= _NCH))
        def _():
            c0 = fire(cid0, idxs0, idxd0, gbuf0, hbuf0, sem0)
            finish(idxd0, gbuf0, hbuf0, c0)

        return carry

    lax.fori_loop(0, (_NT + 1) // 2, pair, 0)
    plsc.subcore_barrier()

    for t in range(_RPT // _ZR):
        base = s * _RPT + t * _ZR
        pltpu.sync_copy(acc.at[pl.ds(base, _ZR)],
                        out_hbm.at[c, pl.ds(base, _ZR)])


_BN = 2000
_BE = 16000


def _enc_body(x_ref, w1, b1, g, be, w2, b2, w1s, w1d, eb1,
              h_ref, a_ref, b_ref):
    h = jnp.dot(x_ref[...], w1[...], preferred_element_type=jnp.float32) + b1[...]
    h = _ln(h, g[...], be[...])
    h = _gelu(h)
    h = jnp.dot(h, w2[...], preferred_element_type=jnp.float32) + b2[...]
    h_ref[...] = jnp.concatenate(
        [h, jnp.ones((h.shape[0], H - D), jnp.float32)], axis=1)
    a_ref[...] = jnp.dot(h, w1s[...], preferred_element_type=jnp.float32) + eb1[...]
    b_ref[...] = jnp.dot(h, w1d[...], preferred_element_type=jnp.float32)


def _encoder(x, w1, b1, g, be, w2, b2, w1s, w1d, eb1):
    grid = (N // _BN,)
    full = lambda shp: pl.BlockSpec(shp, lambda i: (0, 0))
    return pl.pallas_call(
        _enc_body,
        grid=grid,
        in_specs=[
            pl.BlockSpec((_BN, IN), lambda i: (i, 0)),
            full((IN, D)), full((1, D)), full((1, D)), full((1, D)),
            full((D, D)), full((1, D)),
            full((D, H)), full((D, H)), full((1, H)),
        ],
        out_specs=[
            pl.BlockSpec((_BN, H), lambda i: (i, 0)),
            pl.BlockSpec((_BN, H), lambda i: (i, 0)),
            pl.BlockSpec((_BN, H), lambda i: (i, 0)),
        ],
        out_shape=[
            jax.ShapeDtypeStruct((N, H), jnp.float32),
            jax.ShapeDtypeStruct((N, H), jnp.float32),
            jax.ShapeDtypeStruct((N, H), jnp.float32),
        ],
    )(x, w1, b1, g, be, w2, b2, w1s, w1d, eb1)


def _gate_body(p_ref, ea_ref, w1e, w2, b2, gate_ref):
    pre = p_ref[...] + ea_ref[...] * w1e[...]
    gg = _gelu(pre)
    z = jnp.dot(gg, w2[...], preferred_element_type=jnp.float32) + b2[...]
    gate_ref[...] = jax.nn.sigmoid(z)


def _edge_gate(p, ea, w1e, w2, b2):
    grid = (E // _BE,)
    full = lambda shp: pl.BlockSpec(shp, lambda i: (0, 0))
    return pl.pallas_call(
        _gate_body,
        grid=grid,
        in_specs=[
            pl.BlockSpec((_BE, H), lambda i: (i, 0)),
            pl.BlockSpec((_BE, 1), lambda i: (i, 0)),
            full((1, H)), full((H, 1)), full((1, 1)),
        ],
        out_specs=pl.BlockSpec((_BE, 1), lambda i: (i, 0)),
        out_shape=jax.ShapeDtypeStruct((E, 1), jnp.float32),
    )(p, ea, w1e, w2, b2)


def _node_body(h_ref, p0_ref, p1_ref, w1h, w1a, b1, w2, b2, lng, lnb,
               nw1s, nw1d, neb1, h_out, a_out, b_out):
    part = p0_ref[...] + p1_ref[...]
    agg = part[:, :D]
    den = part[:, D:D + 1]
    agg = agg / jnp.maximum(den, 1e-6)
    h = h_ref[:, :D]
    u = _gelu(jnp.dot(h, w1h[...], preferred_element_type=jnp.float32)
              + jnp.dot(agg, w1a[...], preferred_element_type=jnp.float32)
              + b1[...])
    u = jnp.dot(u, w2[...], preferred_element_type=jnp.float32) + b2[...]
    hn = _ln(h + u, lng[...], lnb[...])
    h_out[...] = jnp.concatenate(
        [hn, jnp.ones((hn.shape[0], H - D), jnp.float32)], axis=1)
    a_out[...] = jnp.dot(hn, nw1s[...], preferred_element_type=jnp.float32) + neb1[...]
    b_out[...] = jnp.dot(hn, nw1d[...], preferred_element_type=jnp.float32)


def _node_update(h, p0, p1, w1h, w1a, b1, w2, b2, lng, lnb, nw1s, nw1d, neb1):
    grid = (N // _BN,)
    full = lambda shp: pl.BlockSpec(shp, lambda i: (0, 0))
    return pl.pallas_call(
        _node_body,
        grid=grid,
        in_specs=[
            pl.BlockSpec((_BN, H), lambda i: (i, 0)),
            pl.BlockSpec((_BN, _ACCW), lambda i: (i, 0)),
            pl.BlockSpec((_BN, _ACCW), lambda i: (i, 0)),
            full((D, H)), full((D, H)), full((1, H)),
            full((H, D)), full((1, D)), full((1, D)), full((1, D)),
            full((D, H)), full((D, H)), full((1, H)),
        ],
        out_specs=[
            pl.BlockSpec((_BN, H), lambda i: (i, 0)),
            pl.BlockSpec((_BN, H), lambda i: (i, 0)),
            pl.BlockSpec((_BN, H), lambda i: (i, 0)),
        ],
        out_shape=[
            jax.ShapeDtypeStruct((N, H), jnp.float32),
            jax.ShapeDtypeStruct((N, H), jnp.float32),
            jax.ShapeDtypeStruct((N, H), jnp.float32),
        ],
    )(h, p0, p1, w1h, w1a, b1, w2, b2, lng, lnb, nw1s, nw1d, neb1)


def _head_body(h_ref, p0_ref, p1_ref, w1h, w1a, b1, w2, b2, lng, lnb,
               hw1, hb1, hw2, hb2, out_ref):
    part = p0_ref[...] + p1_ref[...]
    agg = part[:, :D]
    den = part[:, D:D + 1]
    agg = agg / jnp.maximum(den, 1e-6)
    h = h_ref[:, :D]
    u = _gelu(jnp.dot(h, w1h[...], preferred_element_type=jnp.float32)
              + jnp.dot(agg, w1a[...], preferred_element_type=jnp.float32)
              + b1[...])
    u = jnp.dot(u, w2[...], preferred_element_type=jnp.float32) + b2[...]
    hn = _ln(h + u, lng[...], lnb[...])
    raw = jnp.dot(_gelu(jnp.dot(hn, hw1[...], preferred_element_type=jnp.float32)
                        + hb1[...]),
                  hw2[...], preferred_element_type=jnp.float32) + hb2[...]
    out_ref[...] = 1.0 + SCALE * jnp.tanh(raw)


def _head(h, p0, p1, w1h, w1a, b1, w2, b2, lng, lnb, hw1, hb1, hw2, hb2):
    grid = (N // _BN,)
    full = lambda shp: pl.BlockSpec(shp, lambda i: (0, 0))
    return pl.pallas_call(
        _head_body,
        grid=grid,
        in_specs=[
            pl.BlockSpec((_BN, H), lambda i: (i, 0)),
            pl.BlockSpec((_BN, _ACCW), lambda i: (i, 0)),
            pl.BlockSpec((_BN, _ACCW), lambda i: (i, 0)),
            full((D, H)), full((D, H)), full((1, H)),
            full((H, D)), full((1, D)), full((1, D)), full((1, D)),
            full((D, H)), full((1, H)), full((H, 1)), full((1, 1)),
        ],
        out_specs=pl.BlockSpec((_BN, 1), lambda i: (i, 0)),
        out_shape=jax.ShapeDtypeStruct((N, 1), jnp.float32),
    )(h, p0, p1, w1h, w1a, b1, w2, b2, lng, lnb, hw1, hb1, hw2, hb2)


def kernel(x, edge_index, edge_attr, params):
    p = params
    src = jnp.asarray(edge_index[:, 0], jnp.int32)
    dst = jnp.asarray(edge_index[:, 1], jnp.int32)

    def r2(v, n):
        return v.reshape(1, n)

    esplit = []
    for i in range(L):
        w1 = p[f'e{i}_w1']
        esplit.append((w1[:D], w1[D:2 * D], w1[2 * D:2 * D + 1],
                       r2(p[f'e{i}_b1'], H)))

    h, A, B = _encoder(
        x, p['enc_w1'], r2(p['enc_b1'], D), r2(p['enc_g'], D),
        r2(p['enc_be'], D), p['enc_w2'], r2(p['enc_b2'], D),
        esplit[0][0], esplit[0][1], esplit[0][3])

    out = None
    for i in range(L):
        P = _sc_gather_pair(A, B, src, dst)
        gate = _edge_gate(P, edge_attr, esplit[i][2], p[f'e{i}_w2'],
                          r2(p[f'e{i}_b2'], 1))
        parts = _sc_scatter(h, gate.reshape(E), src, dst)
        nw1 = p[f'n{i}_w1']
        common = (h, parts[0], parts[1], nw1[:D], nw1[D:], r2(p[f'n{i}_b1'], H),
                  p[f'n{i}_w2'], r2(p[f'n{i}_b2'], D),
                  r2(p[f'ln{i}_g'], D), r2(p[f'ln{i}_b'], D))
        if i < L - 1:
            h, A, B = _node_update(*common, esplit[i + 1][0], esplit[i + 1][1],
                                   esplit[i + 1][3])
        else:
            out = _head(*common, p['h_w1'], r2(p['h_b1'], H),
                        p['h_w2'], r2(p['h_b2'], 1))
    return out.reshape(N)
